# Initial kernel scaffold; baseline (speedup 1.0000x reference)
import jax
import jax.numpy as jnp
from jax.experimental import pallas as pl


def kernel(input, param):
    raise NotImplementedError("write your pallas kernel here")


# speedup vs baseline: 1.1734x; 1.1734x over previous
import functools

import jax
import jax.numpy as jnp
from jax import lax
from jax.experimental import pallas as pl
from jax.experimental.pallas import tpu as pltpu
from jax.experimental.pallas import tpu_sc as plsc

C = 16
W = 2048
H = 2048
N = 1048576
NW = 32
PER_W = N // NW
CHUNK = 512
NCHUNK = PER_W // CHUNK
L = 16


def _sampler_body(tex_hbm, u_hbm, v_hbm, out_hbm,
                  u_v, v_v,
                  i00, i10, i01, i11,
                  w00, w10, w01, w11,
                  f00, f10, f01, f11,
                  out_v, sem):
    wid = lax.axis_index("s") * 2 + lax.axis_index("c")
    base = wid * PER_W

    def chunk_body(ci, _):
        off = base + ci * CHUNK
        pltpu.sync_copy(u_hbm.at[pl.ds(off, CHUNK)], u_v)
        pltpu.sync_copy(v_hbm.at[pl.ds(off, CHUNK)], v_v)

        def grp_body(gi, _):
            s = pl.ds(gi * L, L)
            u = u_v[s]
            v = v_v[s]
            x = u * jnp.float32(W - 1)
            y = v * jnp.float32(H - 1)
            x0 = jnp.minimum(x.astype(jnp.int32), W - 1)
            y0 = jnp.minimum(y.astype(jnp.int32), H - 1)
            x1 = jnp.minimum(x0 + 1, W - 1)
            y1 = jnp.minimum(y0 + 1, H - 1)
            wx = x - x0.astype(jnp.float32)
            wy = y - y0.astype(jnp.float32)
            omx = 1.0 - wx
            omy = 1.0 - wy
            xr0 = x0 * H
            xr1 = x1 * H
            i00[s] = xr0 + y0
            i10[s] = xr1 + y0
            i01[s] = xr0 + y1
            i11[s] = xr1 + y1
            w00[s] = omx * omy
            w10[s] = wx * omy
            w01[s] = omx * wy
            w11[s] = wx * wy
            return 0

        lax.fori_loop(0, CHUNK // L, grp_body, 0, unroll=2)

        cp0 = pltpu.async_copy(tex_hbm.at[i00], f00, sem)
        cp1 = pltpu.async_copy(tex_hbm.at[i10], f10, sem)
        cp2 = pltpu.async_copy(tex_hbm.at[i01], f01, sem)
        cp3 = pltpu.async_copy(tex_hbm.at[i11], f11, sem)
        cp0.wait()
        cp1.wait()
        cp2.wait()
        cp3.wait()

        def comb_body(gi, _):
            s = pl.ds(gi * L, L)
            w00v = w00[s]
            w10v = w10[s]
            w01v = w01[s]
            w11v = w11[s]
            for k in range(L):
                j = gi * L + k
                acc = (f00[j, :] * w00v[k] + f10[j, :] * w10v[k]
                       + f01[j, :] * w01v[k] + f11[j, :] * w11v[k])
                out_v[j, :] = acc
            return 0

        lax.fori_loop(0, CHUNK // L, comb_body, 0)

        pltpu.sync_copy(out_v, out_hbm.at[pl.ds(off, CHUNK)])
        return 0

    lax.fori_loop(0, NCHUNK, chunk_body, 0)


def kernel(input, param):
    tex = jnp.transpose(input, (1, 2, 0)).reshape(W * H, C)
    u = param[:, 0]
    v = param[:, 1]

    mesh = plsc.VectorSubcoreMesh(core_axis_name="c", subcore_axis_name="s")
    f = pl.kernel(
        _sampler_body,
        out_type=jax.ShapeDtypeStruct((N, C), jnp.float32),
        mesh=mesh,
        compiler_params=pltpu.CompilerParams(use_tc_tiling_on_sc=False),
        scratch_types=[
            pltpu.VMEM((CHUNK,), jnp.float32),
            pltpu.VMEM((CHUNK,), jnp.float32),
            pltpu.VMEM((CHUNK,), jnp.int32),
            pltpu.VMEM((CHUNK,), jnp.int32),
            pltpu.VMEM((CHUNK,), jnp.int32),
            pltpu.VMEM((CHUNK,), jnp.int32),
            pltpu.VMEM((CHUNK,), jnp.float32),
            pltpu.VMEM((CHUNK,), jnp.float32),
            pltpu.VMEM((CHUNK,), jnp.float32),
            pltpu.VMEM((CHUNK,), jnp.float32),
            pltpu.VMEM((CHUNK, C), jnp.float32),
            pltpu.VMEM((CHUNK, C), jnp.float32),
            pltpu.VMEM((CHUNK, C), jnp.float32),
            pltpu.VMEM((CHUNK, C), jnp.float32),
            pltpu.VMEM((CHUNK, C), jnp.float32),
            pltpu.SemaphoreType.DMA,
        ],
    )
    return f(tex, u, v)
